# baseline (speedup 1.0000x reference)
import functools

import jax
import jax.numpy as jnp
from jax import lax
from jax.experimental import pallas as pl
from jax.experimental.pallas import tpu as pltpu
from jax.experimental.pallas import tpu_sc as plsc

EMBED = 128
SEQ = 50
SC_K = 4
NBUF = 4
AHEAD = NBUF - 2
NCHUNK = 4
RB = 8


def _make_gather2d(num_rows: int):
  info = plsc.get_sparse_core_info()
  nc, ns = info.num_cores, info.num_subcores
  nw = nc * ns
  assert num_rows % (nw * SC_K) == 0
  rows_per_w = num_rows // nw
  T = rows_per_w // SC_K
  blk = SC_K * SEQ

  mesh = plsc.VectorSubcoreMesh(core_axis_name="c", subcore_axis_name="s")

  @functools.partial(
      pl.kernel,
      mesh=mesh,
      out_type=jax.ShapeDtypeStruct((num_rows * SEQ, EMBED), jnp.float32),
      scratch_types=(
          [pltpu.VMEM((rows_per_w, SEQ), jnp.int32)]
          + [pltpu.VMEM((blk, EMBED), jnp.float32) for _ in range(NBUF)]
          + [pltpu.SemaphoreType.DMA for _ in range(2 * NBUF)]
      ),
  )
  def gather_kernel(idx_hbm, table_hbm, out_hbm, idx_v, *rest):
    bufs = rest[:NBUF]
    gsem = rest[NBUF:2 * NBUF]
    ssem = rest[2 * NBUF:]
    wid = lax.axis_index("s") * nc + lax.axis_index("c")
    pltpu.sync_copy(idx_hbm.at[wid], idx_v)
    base = wid * rows_per_w * SEQ

    def g_start(b, j):
      for i in range(SC_K):
        pltpu.async_copy(
            table_hbm.at[idx_v.at[j * SC_K + i]],
            bufs[b].at[pl.ds(i * SEQ, SEQ)], gsem[b])

    def g_wait(b):
      pltpu.make_async_copy(
          out_hbm.at[pl.ds(0, blk)], bufs[b], gsem[b]).wait()

    def s_start(b, j):
      off = pl.multiple_of(base + j * blk, 8)
      pltpu.async_copy(bufs[b], out_hbm.at[pl.ds(off, blk)], ssem[b])

    def s_wait(b):
      pltpu.make_async_copy(
          bufs[b], out_hbm.at[pl.ds(0, blk)], ssem[b]).wait()

    for j in range(AHEAD):
      g_start(j % NBUF, j)
    for j in range(2):
      g_start((j + AHEAD) % NBUF, j + AHEAD)
      g_wait(j % NBUF)
      s_start(j % NBUF, j)

    main_lo, main_hi = 2, T - AHEAD
    n_iters = main_hi - main_lo
    n_outer = n_iters // NBUF
    n_rem = n_iters % NBUF

    def outer(t, carry):
      for i in range(NBUF):
        j = main_lo + t * NBUF + i
        b = (main_lo + i) % NBUF
        s_wait((b - 2) % NBUF)
        g_start((b + AHEAD) % NBUF, j + AHEAD)
        g_wait(b)
        s_start(b, j)
      return carry

    lax.fori_loop(0, n_outer, outer, 0)
    for k in range(n_rem):
      j = main_lo + n_outer * NBUF + k
      b = (main_lo + k) % NBUF
      s_wait((b - 2) % NBUF)
      g_start((b + AHEAD) % NBUF, j + AHEAD)
      g_wait(b)
      s_start(b, j)
    for j in range(T - AHEAD, T):
      b = j % NBUF
      s_wait((b - 2) % NBUF)
      g_wait(b)
      s_start(b, j)
    s_wait((T - 2) % NBUF)
    s_wait((T - 1) % NBUF)

  return gather_kernel


def _relayout_body(chunk_ref, out_ref):
  for r in range(RB):
    out_ref[r] = chunk_ref[pl.ds(r * SEQ, SEQ), :]


def _relayout_first(chunk, num_rows, rows_c):
  grid = rows_c // RB
  return pl.pallas_call(
      _relayout_body,
      grid=(grid,),
      in_specs=[pl.BlockSpec((RB * SEQ, EMBED), lambda i: (i, 0))],
      out_specs=pl.BlockSpec((RB, SEQ, EMBED), lambda i: (i, 0, 0)),
      out_shape=jax.ShapeDtypeStruct((num_rows, SEQ, EMBED), jnp.float32),
  )(chunk)


def _relayout_update(acc, chunk, c, rows_c):
  grid = rows_c // RB
  blk0 = c * rows_c // RB

  def body(acc_ref, chunk_ref, out_ref):
    _relayout_body(chunk_ref, out_ref)

  return pl.pallas_call(
      body,
      grid=(grid,),
      in_specs=[
          pl.BlockSpec(memory_space=pl.ANY),
          pl.BlockSpec((RB * SEQ, EMBED), lambda i: (i, 0)),
      ],
      out_specs=pl.BlockSpec((RB, SEQ, EMBED), lambda i: (blk0 + i, 0, 0)),
      out_shape=jax.ShapeDtypeStruct(acc.shape, jnp.float32),
      input_output_aliases={0: 0},
  )(acc, chunk)


def kernel(x, table):
  num_rows, seq = x.shape
  assert seq == SEQ
  info = plsc.get_sparse_core_info()
  nw = info.num_cores * info.num_subcores
  rows_c = num_rows // NCHUNK
  gather = _make_gather2d(rows_c)
  xi = x.astype(jnp.int32)
  chunks = []
  for c in range(NCHUNK):
    idx3 = xi[c * rows_c:(c + 1) * rows_c].reshape(nw, rows_c // nw, SEQ)
    chunks.append(gather(idx3, table))
  acc = _relayout_first(chunks[0], num_rows, rows_c)
  for c in range(1, NCHUNK):
    acc = _relayout_update(acc, chunks[c], c, rows_c)
  return acc


# speedup vs baseline: 2.7113x; 2.7113x over previous
import functools

import jax
import jax.numpy as jnp
from jax import lax
from jax.experimental import pallas as pl
from jax.experimental.pallas import tpu as pltpu
from jax.experimental.pallas import tpu_sc as plsc

EMBED = 128
SEQ = 50
SC_K = 2
NBUF = 8
AHEAD = NBUF - 2


def _make_gather(num_rows: int):
  info = plsc.get_sparse_core_info()
  nc, ns = info.num_cores, info.num_subcores
  nw = nc * ns
  assert num_rows % (nw * SC_K) == 0
  rows_per_w = num_rows // nw
  T = rows_per_w // SC_K

  mesh = plsc.VectorSubcoreMesh(core_axis_name="c", subcore_axis_name="s")

  @functools.partial(
      pl.kernel,
      mesh=mesh,
      out_type=jax.ShapeDtypeStruct((num_rows, SEQ, EMBED), jnp.float32),
      scratch_types=(
          [pltpu.VMEM((rows_per_w, SEQ), jnp.int32)]
          + [pltpu.VMEM((SC_K, SEQ, EMBED), jnp.float32) for _ in range(NBUF)]
          + [pltpu.SemaphoreType.DMA for _ in range(2 * NBUF)]
      ),
  )
  def gather_kernel(idx_hbm, table_hbm, out_hbm, idx_v, *rest):
    bufs = rest[:NBUF]
    gsem = rest[NBUF:2 * NBUF]
    ssem = rest[2 * NBUF:]
    wid = lax.axis_index("s") * nc + lax.axis_index("c")
    pltpu.sync_copy(idx_hbm.at[wid], idx_v)
    row0 = wid * rows_per_w

    def g_start(b, j):
      for i in range(SC_K):
        pltpu.async_copy(
            table_hbm.at[idx_v.at[j * SC_K + i]], bufs[b].at[i], gsem[b])

    def g_wait(b):
      pltpu.make_async_copy(
          out_hbm.at[pl.ds(0, SC_K)], bufs[b], gsem[b]).wait()

    def s_start(b, j):
      pltpu.async_copy(
          bufs[b], out_hbm.at[pl.ds(row0 + j * SC_K, SC_K)], ssem[b])

    def s_wait(b):
      pltpu.make_async_copy(
          bufs[b], out_hbm.at[pl.ds(0, SC_K)], ssem[b]).wait()

    for j in range(AHEAD):
      g_start(j % NBUF, j)
    for j in range(2):
      g_start((j + AHEAD) % NBUF, j + AHEAD)
      g_wait(j % NBUF)
      s_start(j % NBUF, j)

    main_lo, main_hi = 2, T - AHEAD
    n_iters = main_hi - main_lo
    n_outer = n_iters // NBUF
    n_rem = n_iters % NBUF

    def outer(t, carry):
      for i in range(NBUF):
        j = main_lo + t * NBUF + i
        b = (main_lo + i) % NBUF
        s_wait((b - 2) % NBUF)
        g_start((b + AHEAD) % NBUF, j + AHEAD)
        g_wait(b)
        s_start(b, j)
      return carry

    lax.fori_loop(0, n_outer, outer, 0)
    for k in range(n_rem):
      j = main_lo + n_outer * NBUF + k
      b = (main_lo + k) % NBUF
      s_wait((b - 2) % NBUF)
      g_start((b + AHEAD) % NBUF, j + AHEAD)
      g_wait(b)
      s_start(b, j)
    for j in range(T - AHEAD, T):
      b = j % NBUF
      s_wait((b - 2) % NBUF)
      g_wait(b)
      s_start(b, j)
    s_wait((T - 2) % NBUF)
    s_wait((T - 1) % NBUF)

  return gather_kernel


def kernel(x, table):
  num_rows, seq = x.shape
  assert seq == SEQ
  info = plsc.get_sparse_core_info()
  nw = info.num_cores * info.num_subcores
  idx3 = x.astype(jnp.int32).reshape(nw, num_rows // nw, SEQ)
  return _make_gather(num_rows)(idx3, table)
